# baseline (speedup 1.0000x reference)
import functools

import jax
import jax.numpy as jnp
from jax import lax
from jax.experimental import pallas as pl
from jax.experimental.pallas import tpu as pltpu

N_TOKENS = 4096
D_MODEL = 2048
D_FF = 2048
N_TASKS = 8

TILE = 256
NT = N_TOKENS // TILE
NUM_ITEMS = NT + N_TASKS - 1


def _routing(tasks):
    t32 = tasks.astype(jnp.int32)
    sort_idx = jnp.argsort(t32).astype(jnp.int32)
    inv_idx = jnp.zeros((N_TOKENS,), jnp.int32).at[sort_idx].set(
        jnp.arange(N_TOKENS, dtype=jnp.int32))
    sorted_tasks = t32[sort_idx]
    offsets = jnp.searchsorted(
        sorted_tasks, jnp.arange(N_TASKS + 1, dtype=jnp.int32), side="left"
    ).astype(jnp.int32)
    st = sorted_tasks.reshape(NT, TILE)
    lo = st[:, 0]
    hi = st[:, -1]
    n_items = hi - lo + 1
    starts = jnp.concatenate(
        [jnp.zeros((1,), jnp.int32), jnp.cumsum(n_items, dtype=jnp.int32)])
    total = starts[NT]
    ii = jnp.arange(NUM_ITEMS, dtype=jnp.int32)
    t_of_i = jnp.clip(jnp.searchsorted(starts, ii, side="right") - 1, 0, NT - 1)
    e_of_i = jnp.clip(lo[t_of_i] + ii - starts[t_of_i], 0, N_TASKS - 1)
    pad = ii >= total
    tile_ids = jnp.where(pad, NT - 1, t_of_i)
    group_ids = jnp.where(pad, hi[NT - 1], e_of_i)
    return sort_idx, inv_idx, offsets, tile_ids, group_ids


def _mlp_item(ti_ref, gi_ref, off_ref,
              x_ref, w1_ref, b1_ref, w2_ref, b2_ref, out_ref):
    i = pl.program_id(0)
    e = gi_ref[i]
    t = ti_ref[i]
    x = x_ref[...]
    h = jnp.dot(x, w1_ref[0], preferred_element_type=jnp.float32)
    h = jnp.maximum(h + b1_ref[pl.ds(e, 1), :], 0.0)
    y = jnp.dot(h.astype(jnp.bfloat16), w2_ref[0],
                preferred_element_type=jnp.float32)
    y = y + b2_ref[pl.ds(e, 1), :]
    rows = t * TILE + lax.broadcasted_iota(jnp.int32, (TILE, 1), 0)
    mask = (rows >= off_ref[e]) & (rows < off_ref[e + 1])
    out_ref[...] = jnp.where(mask, y, out_ref[...])


def _grouped_mlp(x_sorted, W1, b1, W2, b2, offsets, tile_ids, group_ids):
    grid_spec = pltpu.PrefetchScalarGridSpec(
        num_scalar_prefetch=3,
        grid=(NUM_ITEMS,),
        in_specs=[
            pl.BlockSpec((TILE, D_MODEL), lambda i, ti, gi, off: (ti[i], 0)),
            pl.BlockSpec((1, D_MODEL, D_FF), lambda i, ti, gi, off: (gi[i], 0, 0)),
            pl.BlockSpec((N_TASKS, D_FF), lambda i, ti, gi, off: (0, 0)),
            pl.BlockSpec((1, D_FF, D_MODEL), lambda i, ti, gi, off: (gi[i], 0, 0)),
            pl.BlockSpec((N_TASKS, D_MODEL), lambda i, ti, gi, off: (0, 0)),
        ],
        out_specs=pl.BlockSpec((TILE, D_MODEL), lambda i, ti, gi, off: (ti[i], 0)),
    )
    return pl.pallas_call(
        _mlp_item,
        grid_spec=grid_spec,
        out_shape=jax.ShapeDtypeStruct((N_TOKENS, D_MODEL), jnp.float32),
        compiler_params=pltpu.CompilerParams(
            dimension_semantics=("arbitrary",)),
    )(tile_ids, group_ids, offsets, x_sorted,
      W1.astype(jnp.bfloat16), b1, W2.astype(jnp.bfloat16), b2)


def kernel(h_root, tasks, W1, b1, W2, b2):
    sort_idx, inv_idx, offsets, tile_ids, group_ids = _routing(tasks)
    x_sorted = jnp.take(h_root, sort_idx, axis=0).astype(jnp.bfloat16)
    y_sorted = _grouped_mlp(x_sorted, W1, b1, W2, b2,
                            offsets, tile_ids, group_ids)
    return jnp.take(y_sorted, inv_idx, axis=0)


# speedup vs baseline: 1.2791x; 1.2791x over previous
import functools

import jax
import jax.numpy as jnp
from jax import lax
from jax.experimental import pallas as pl
from jax.experimental.pallas import tpu as pltpu

N_TOKENS = 4096
D_MODEL = 2048
D_FF = 2048
N_TASKS = 8

TILE = 256
NT = N_TOKENS // TILE
NUM_ITEMS = NT + N_TASKS - 1


def _routing(tasks):
    t32 = tasks.astype(jnp.int32)
    sort_idx = jnp.argsort(t32).astype(jnp.int32)
    inv_idx = jnp.zeros((N_TOKENS,), jnp.int32).at[sort_idx].set(
        jnp.arange(N_TOKENS, dtype=jnp.int32))
    sorted_tasks = t32[sort_idx]
    offsets = jnp.searchsorted(
        sorted_tasks, jnp.arange(N_TASKS + 1, dtype=jnp.int32), side="left"
    ).astype(jnp.int32)
    st = sorted_tasks.reshape(NT, TILE)
    lo = st[:, 0]
    hi = st[:, -1]
    n_items = hi - lo + 1
    starts = jnp.concatenate(
        [jnp.zeros((1,), jnp.int32), jnp.cumsum(n_items, dtype=jnp.int32)])
    total = starts[NT]
    ii = jnp.arange(NUM_ITEMS, dtype=jnp.int32)
    t_of_i = jnp.clip(jnp.searchsorted(starts, ii, side="right") - 1, 0, NT - 1)
    e_of_i = jnp.clip(lo[t_of_i] + ii - starts[t_of_i], 0, N_TASKS - 1)
    pad = ii >= total
    tile_ids = jnp.where(pad, NT - 1, t_of_i)
    group_ids = jnp.where(pad, hi[NT - 1], e_of_i)
    return sort_idx, inv_idx, offsets, tile_ids, group_ids


def _mlp_item(ti_ref, gi_ref, off_ref,
              x_ref, w1_ref, b1_ref, w2_hbm, b2_ref, out_ref,
              w2_scr, sem):
    i = pl.program_id(0)
    e = gi_ref[i]
    t = ti_ref[i]
    prev_e = gi_ref[jnp.maximum(i - 1, 0)]
    is_first = jnp.logical_or(i == 0, e != prev_e)

    @pl.when(is_first)
    def _load_w2():
        cp = pltpu.make_async_copy(w2_hbm.at[e], w2_scr, sem)
        cp.start()
        cp.wait()

    x = x_ref[...]
    h = jnp.dot(x, w1_ref[0], preferred_element_type=jnp.float32)
    h = jnp.maximum(h + b1_ref[pl.ds(e, 1), :], 0.0)
    y = jnp.dot(h, w2_scr[...], preferred_element_type=jnp.float32)
    y = y + b2_ref[pl.ds(e, 1), :]
    rows = t * TILE + lax.broadcasted_iota(jnp.int32, (TILE, 1), 0)
    mask = (rows >= off_ref[e]) & (rows < off_ref[e + 1])
    out_ref[...] = jnp.where(mask, y, out_ref[...])


def _grouped_mlp(x_sorted, W1, b1, W2, b2, offsets, tile_ids, group_ids):
    grid_spec = pltpu.PrefetchScalarGridSpec(
        num_scalar_prefetch=3,
        grid=(NUM_ITEMS,),
        in_specs=[
            pl.BlockSpec((TILE, D_MODEL), lambda i, ti, gi, off: (ti[i], 0)),
            pl.BlockSpec((1, D_MODEL, D_FF), lambda i, ti, gi, off: (gi[i], 0, 0)),
            pl.BlockSpec((N_TASKS, D_FF), lambda i, ti, gi, off: (0, 0)),
            pl.BlockSpec(memory_space=pltpu.MemorySpace.HBM),
            pl.BlockSpec((N_TASKS, D_MODEL), lambda i, ti, gi, off: (0, 0)),
        ],
        out_specs=pl.BlockSpec((TILE, D_MODEL), lambda i, ti, gi, off: (ti[i], 0)),
        scratch_shapes=[
            pltpu.VMEM((D_FF, D_MODEL), jnp.float32),
            pltpu.SemaphoreType.DMA,
        ],
    )
    return pl.pallas_call(
        _mlp_item,
        grid_spec=grid_spec,
        out_shape=jax.ShapeDtypeStruct((N_TOKENS, D_MODEL), jnp.float32),
        compiler_params=pltpu.CompilerParams(
            dimension_semantics=("arbitrary",)),
    )(tile_ids, group_ids, offsets, x_sorted, W1, b1, W2, b2)


def kernel(h_root, tasks, W1, b1, W2, b2):
    sort_idx, inv_idx, offsets, tile_ids, group_ids = _routing(tasks)
    x_sorted = jnp.take(h_root, sort_idx, axis=0)
    y_sorted = _grouped_mlp(x_sorted, W1, b1, W2, b2,
                            offsets, tile_ids, group_ids)
    return jnp.take(y_sorted, inv_idx, axis=0)


# speedup vs baseline: 1.3956x; 1.0911x over previous
import functools

import jax
import jax.numpy as jnp
from jax import lax
from jax.experimental import pallas as pl
from jax.experimental.pallas import tpu as pltpu

N_TOKENS = 4096
D_MODEL = 2048
D_FF = 2048
N_TASKS = 8

TILE = 256
NT = N_TOKENS // TILE
NUM_ITEMS = NT + N_TASKS - 1


def _routing(tasks):
    t32 = tasks.astype(jnp.int32)
    oh = (t32[:, None] == jnp.arange(N_TASKS, dtype=jnp.int32)[None, :]
          ).astype(jnp.int32)
    counts = jnp.sum(oh, axis=0)
    offsets = jnp.concatenate([jnp.zeros((1,), jnp.int32),
                               jnp.cumsum(counts, dtype=jnp.int32)])
    pos = jnp.cumsum(oh, axis=0)
    rank_within = jnp.sum(oh * pos, axis=1) - 1
    inv_idx = offsets[t32] + rank_within
    sort_idx = jnp.zeros((N_TOKENS,), jnp.int32).at[inv_idx].set(
        jnp.arange(N_TOKENS, dtype=jnp.int32))
    tile_starts = jnp.arange(NT, dtype=jnp.int32) * TILE
    lo = jnp.sum(offsets[None, 1:N_TASKS] <= tile_starts[:, None], axis=1)
    hi = jnp.sum(offsets[None, 1:N_TASKS] <= (tile_starts + TILE - 1)[:, None],
                 axis=1)
    n_items = hi - lo + 1
    starts = jnp.concatenate(
        [jnp.zeros((1,), jnp.int32), jnp.cumsum(n_items, dtype=jnp.int32)])
    total = starts[NT]
    ii = jnp.arange(NUM_ITEMS, dtype=jnp.int32)
    t_of_i = jnp.clip(jnp.searchsorted(starts, ii, side="right") - 1, 0, NT - 1)
    e_of_i = jnp.clip(lo[t_of_i] + ii - starts[t_of_i], 0, N_TASKS - 1)
    pad = ii >= total
    tile_ids = jnp.where(pad, NT - 1, t_of_i)
    group_ids = jnp.where(pad, hi[NT - 1], e_of_i)
    return sort_idx, inv_idx, offsets, tile_ids, group_ids


def _mlp_item(ti_ref, gi_ref, off_ref,
              x_ref, w1_ref, b1_ref, w2_hbm, b2_ref, out_ref,
              w2_scr, sem):
    i = pl.program_id(0)
    e = gi_ref[i]
    t = ti_ref[i]
    prev_e = gi_ref[jnp.maximum(i - 1, 0)]
    next_e = gi_ref[jnp.minimum(i + 1, NUM_ITEMS - 1)]
    is_first = jnp.logical_or(i == 0, e != prev_e)

    @pl.when(i == 0)
    def _load_w2_first():
        pltpu.make_async_copy(w2_hbm.at[e], w2_scr, sem).start()

    x = x_ref[...]
    h = jnp.dot(x, w1_ref[0], preferred_element_type=jnp.float32)
    h = jnp.maximum(h + b1_ref[pl.ds(e, 1), :], 0.0)

    @pl.when(is_first)
    def _wait_w2():
        pltpu.make_async_copy(w2_hbm.at[e], w2_scr, sem).wait()

    y = jnp.dot(h, w2_scr[...], preferred_element_type=jnp.float32)
    y = y + b2_ref[pl.ds(e, 1), :]
    rows = t * TILE + lax.broadcasted_iota(jnp.int32, (TILE, 1), 0)
    mask = (rows >= off_ref[e]) & (rows < off_ref[e + 1])
    out_ref[...] = jnp.where(mask, y, out_ref[...])

    @pl.when(next_e != e)
    def _prefetch_w2():
        pltpu.make_async_copy(w2_hbm.at[next_e], w2_scr, sem).start()


def _grouped_mlp(x_sorted, W1, b1, W2, b2, offsets, tile_ids, group_ids):
    grid_spec = pltpu.PrefetchScalarGridSpec(
        num_scalar_prefetch=3,
        grid=(NUM_ITEMS,),
        in_specs=[
            pl.BlockSpec((TILE, D_MODEL), lambda i, ti, gi, off: (ti[i], 0)),
            pl.BlockSpec((1, D_MODEL, D_FF), lambda i, ti, gi, off: (gi[i], 0, 0)),
            pl.BlockSpec((N_TASKS, D_FF), lambda i, ti, gi, off: (0, 0)),
            pl.BlockSpec(memory_space=pltpu.MemorySpace.HBM),
            pl.BlockSpec((N_TASKS, D_MODEL), lambda i, ti, gi, off: (0, 0)),
        ],
        out_specs=pl.BlockSpec((TILE, D_MODEL), lambda i, ti, gi, off: (ti[i], 0)),
        scratch_shapes=[
            pltpu.VMEM((D_FF, D_MODEL), jnp.float32),
            pltpu.SemaphoreType.DMA,
        ],
    )
    return pl.pallas_call(
        _mlp_item,
        grid_spec=grid_spec,
        out_shape=jax.ShapeDtypeStruct((N_TOKENS, D_MODEL), jnp.float32),
        compiler_params=pltpu.CompilerParams(
            dimension_semantics=("arbitrary",)),
    )(tile_ids, group_ids, offsets, x_sorted, W1, b1, W2, b2)


def kernel(h_root, tasks, W1, b1, W2, b2):
    sort_idx, inv_idx, offsets, tile_ids, group_ids = _routing(tasks)
    x_sorted = jnp.take(h_root, sort_idx, axis=0)
    y_sorted = _grouped_mlp(x_sorted, W1, b1, W2, b2,
                            offsets, tile_ids, group_ids)
    return jnp.take(y_sorted, inv_idx, axis=0)


# speedup vs baseline: 1.6876x; 1.2092x over previous
import functools

import jax
import jax.numpy as jnp
from jax import lax
from jax.experimental import pallas as pl
from jax.experimental.pallas import tpu as pltpu
from jax.experimental.pallas import tpu_sc as plsc

N_TOKENS = 4096
D_MODEL = 2048
D_FF = 2048
N_TASKS = 8

TILE = 256
NT = N_TOKENS // TILE
NUM_ITEMS = NT + N_TASKS - 1


def _routing(tasks):
    t32 = tasks.astype(jnp.int32)
    oh = (t32[:, None] == jnp.arange(N_TASKS, dtype=jnp.int32)[None, :]
          ).astype(jnp.int32)
    counts = jnp.sum(oh, axis=0)
    offsets = jnp.concatenate([jnp.zeros((1,), jnp.int32),
                               jnp.cumsum(counts, dtype=jnp.int32)])
    pos = jnp.cumsum(oh, axis=0)
    rank_within = jnp.sum(oh * pos, axis=1) - 1
    inv_idx = offsets[t32] + rank_within
    sort_idx = jnp.zeros((N_TOKENS,), jnp.int32).at[inv_idx].set(
        jnp.arange(N_TOKENS, dtype=jnp.int32))
    tile_starts = jnp.arange(NT, dtype=jnp.int32) * TILE
    lo = jnp.sum(offsets[None, 1:N_TASKS] <= tile_starts[:, None], axis=1)
    hi = jnp.sum(offsets[None, 1:N_TASKS] <= (tile_starts + TILE - 1)[:, None],
                 axis=1)
    n_items = hi - lo + 1
    starts = jnp.concatenate(
        [jnp.zeros((1,), jnp.int32), jnp.cumsum(n_items, dtype=jnp.int32)])
    total = starts[NT]
    ii = jnp.arange(NUM_ITEMS, dtype=jnp.int32)
    t_of_i = jnp.clip(jnp.searchsorted(starts, ii, side="right") - 1, 0, NT - 1)
    e_of_i = jnp.clip(lo[t_of_i] + ii - starts[t_of_i], 0, N_TASKS - 1)
    pad = ii >= total
    tile_ids = jnp.where(pad, NT - 1, t_of_i)
    group_ids = jnp.where(pad, hi[NT - 1], e_of_i)
    return sort_idx, inv_idx, offsets, tile_ids, group_ids


def _mlp_item(ti_ref, gi_ref, off_ref,
              x_ref, w1_ref, b1_ref, w2_hbm, b2_ref, out_ref,
              w2_scr, sem):
    i = pl.program_id(0)
    e = gi_ref[i]
    t = ti_ref[i]
    prev_e = gi_ref[jnp.maximum(i - 1, 0)]
    next_e = gi_ref[jnp.minimum(i + 1, NUM_ITEMS - 1)]
    is_first = jnp.logical_or(i == 0, e != prev_e)

    @pl.when(i == 0)
    def _load_w2_first():
        pltpu.make_async_copy(w2_hbm.at[e], w2_scr, sem).start()

    x = x_ref[...]
    h = jnp.dot(x, w1_ref[0], preferred_element_type=jnp.float32)
    h = jnp.maximum(h + b1_ref[pl.ds(e, 1), :], 0.0)

    @pl.when(is_first)
    def _wait_w2():
        pltpu.make_async_copy(w2_hbm.at[e], w2_scr, sem).wait()

    y = jnp.dot(h, w2_scr[...], preferred_element_type=jnp.float32)
    y = y + b2_ref[pl.ds(e, 1), :]
    rows = t * TILE + lax.broadcasted_iota(jnp.int32, (TILE, 1), 0)
    mask = (rows >= off_ref[e]) & (rows < off_ref[e + 1])
    out_ref[...] = jnp.where(mask, y, out_ref[...])

    @pl.when(next_e != e)
    def _prefetch_w2():
        pltpu.make_async_copy(w2_hbm.at[next_e], w2_scr, sem).start()


def _grouped_mlp(x_sorted, W1, b1, W2, b2, offsets, tile_ids, group_ids):
    grid_spec = pltpu.PrefetchScalarGridSpec(
        num_scalar_prefetch=3,
        grid=(NUM_ITEMS,),
        in_specs=[
            pl.BlockSpec((TILE, D_MODEL), lambda i, ti, gi, off: (ti[i], 0)),
            pl.BlockSpec((1, D_MODEL, D_FF), lambda i, ti, gi, off: (gi[i], 0, 0)),
            pl.BlockSpec((N_TASKS, D_FF), lambda i, ti, gi, off: (0, 0)),
            pl.BlockSpec(memory_space=pltpu.MemorySpace.HBM),
            pl.BlockSpec((N_TASKS, D_MODEL), lambda i, ti, gi, off: (0, 0)),
        ],
        out_specs=pl.BlockSpec((TILE, D_MODEL), lambda i, ti, gi, off: (ti[i], 0)),
        scratch_shapes=[
            pltpu.VMEM((D_FF, D_MODEL), jnp.float32),
            pltpu.SemaphoreType.DMA,
        ],
    )
    return pl.pallas_call(
        _mlp_item,
        grid_spec=grid_spec,
        out_shape=jax.ShapeDtypeStruct((N_TOKENS, D_MODEL), jnp.float32),
        compiler_params=pltpu.CompilerParams(
            dimension_semantics=("arbitrary",)),
    )(tile_ids, group_ids, offsets, x_sorted, W1, b1, W2, b2)


_SC_INFO = plsc.get_sparse_core_info()
_NC = _SC_INFO.num_cores
_NS = _SC_INFO.num_subcores
_NW = _NC * _NS
_BPW = N_TOKENS // _NW
_CH = 16
_NCH = _BPW // _CH


def _sc_gather_rows(table, idx):
    mesh = plsc.VectorSubcoreMesh(core_axis_name="c", subcore_axis_name="s")

    @functools.partial(
        pl.kernel, mesh=mesh,
        out_type=jax.ShapeDtypeStruct((N_TOKENS, D_MODEL), jnp.float32),
        scratch_types=[
            pltpu.VMEM((_BPW,), jnp.int32),
            pltpu.VMEM((2, _CH, D_MODEL), jnp.float32),
            pltpu.SemaphoreType.DMA,
            pltpu.SemaphoreType.DMA,
            pltpu.SemaphoreType.DMA,
            pltpu.SemaphoreType.DMA,
        ],
    )
    def gather_kernel(table_hbm, idx_hbm, out_hbm, idx_v, bufs, g0, g1, w0, w1):
        wid = lax.axis_index("s") * _NC + lax.axis_index("c")
        base = wid * _BPW
        pltpu.sync_copy(idx_hbm.at[pl.ds(base, _BPW)], idx_v)
        gs = (g0, g1)
        ws = (w0, w1)

        def g_copy(c):
            return pltpu.make_async_copy(
                table_hbm.at[idx_v.at[pl.ds(c * _CH, _CH)]],
                bufs.at[c % 2], gs[c % 2])

        def w_copy(c):
            return pltpu.make_async_copy(
                bufs.at[c % 2], out_hbm.at[pl.ds(base + c * _CH, _CH)],
                ws[c % 2])

        g_copy(0).start()
        for c in range(_NCH):
            if c + 1 < _NCH:
                if c >= 1:
                    w_copy(c - 1).wait()
                g_copy(c + 1).start()
            g_copy(c).wait()
            w_copy(c).start()
        w_copy(_NCH - 2).wait()
        w_copy(_NCH - 1).wait()

    return gather_kernel(table, idx)


def kernel(h_root, tasks, W1, b1, W2, b2):
    sort_idx, inv_idx, offsets, tile_ids, group_ids = _routing(tasks)
    x_sorted = _sc_gather_rows(h_root, sort_idx)
    y_sorted = _grouped_mlp(x_sorted, W1, b1, W2, b2,
                            offsets, tile_ids, group_ids)
    return _sc_gather_rows(y_sorted, inv_idx)


# speedup vs baseline: 1.7092x; 1.0128x over previous
import functools

import jax
import jax.numpy as jnp
from jax import lax
from jax.experimental import pallas as pl
from jax.experimental.pallas import tpu as pltpu
from jax.experimental.pallas import tpu_sc as plsc

N_TOKENS = 4096
D_MODEL = 2048
D_FF = 2048
N_TASKS = 8

TILE = 256
NT = N_TOKENS // TILE
NUM_ITEMS = NT + N_TASKS - 1


def _routing(tasks):
    t32 = tasks.astype(jnp.int32)
    oh = (t32[:, None] == jnp.arange(N_TASKS, dtype=jnp.int32)[None, :]
          ).astype(jnp.int16)
    counts = jnp.sum(oh.astype(jnp.int32), axis=0)
    offsets = jnp.concatenate([jnp.zeros((1,), jnp.int32),
                               jnp.cumsum(counts, dtype=jnp.int32)])
    pos = jnp.cumsum(oh, axis=0, dtype=jnp.int16)
    rank_within = jnp.sum(oh * pos, axis=1, dtype=jnp.int32) - 1
    inv_idx = offsets[t32] + rank_within
    sort_idx = jnp.zeros((N_TOKENS,), jnp.int32).at[inv_idx].set(
        jnp.arange(N_TOKENS, dtype=jnp.int32))
    tile_starts = jnp.arange(NT, dtype=jnp.int32) * TILE
    lo = jnp.sum(offsets[None, 1:N_TASKS] <= tile_starts[:, None], axis=1)
    hi = jnp.sum(offsets[None, 1:N_TASKS] <= (tile_starts + TILE - 1)[:, None],
                 axis=1)
    n_items = hi - lo + 1
    starts = jnp.concatenate(
        [jnp.zeros((1,), jnp.int32), jnp.cumsum(n_items, dtype=jnp.int32)])
    total = starts[NT]
    ii = jnp.arange(NUM_ITEMS, dtype=jnp.int32)
    t_of_i = jnp.clip(jnp.searchsorted(starts, ii, side="right") - 1, 0, NT - 1)
    e_of_i = jnp.clip(lo[t_of_i] + ii - starts[t_of_i], 0, N_TASKS - 1)
    pad = ii >= total
    tile_ids = jnp.where(pad, NT - 1, t_of_i)
    group_ids = jnp.where(pad, hi[NT - 1], e_of_i)
    run_start = jnp.concatenate(
        [jnp.ones((1,), jnp.int32),
         (group_ids[1:] != group_ids[:-1]).astype(jnp.int32)])
    parity = (jnp.cumsum(run_start) - 1) % 2
    later = (group_ids[None, :] != group_ids[:, None]) & (ii[None, :] > ii[:, None])
    has_nxt = jnp.any(later, axis=1).astype(jnp.int32)
    nxt_e = jnp.where(has_nxt == 1,
                      group_ids[jnp.argmax(later, axis=1)], group_ids)
    return (sort_idx, inv_idx, offsets, tile_ids, group_ids,
            parity.astype(jnp.int32), nxt_e.astype(jnp.int32), has_nxt)


def _mlp_item(ti_ref, gi_ref, off_ref, par_ref, nxt_ref, hn_ref,
              x_ref, w1_hbm, b1_ref, w2_hbm, b2_ref, out_ref,
              w1_scr, w2_scr, w1_sem, w2_sem):
    i = pl.program_id(0)
    e = gi_ref[i]
    t = ti_ref[i]
    p = par_ref[i]
    prev_e = gi_ref[jnp.maximum(i - 1, 0)]
    next_e = gi_ref[jnp.minimum(i + 1, NUM_ITEMS - 1)]
    is_first = jnp.logical_or(i == 0, e != prev_e)

    @pl.when(i == 0)
    def _load_first():
        pltpu.make_async_copy(w1_hbm.at[e], w1_scr.at[p], w1_sem).start()
        pltpu.make_async_copy(w2_hbm.at[e], w2_scr, w2_sem).start()

    @pl.when(is_first)
    def _wait_w1():
        pltpu.make_async_copy(w1_hbm.at[e], w1_scr.at[p], w1_sem).wait()

    @pl.when(is_first & (hn_ref[i] == 1))
    def _prefetch_w1():
        pltpu.make_async_copy(
            w1_hbm.at[nxt_ref[i]], w1_scr.at[1 - p], w1_sem).start()

    x = x_ref[...]
    h = jnp.dot(x, w1_scr[p], preferred_element_type=jnp.float32)
    h = jnp.maximum(h + b1_ref[pl.ds(e, 1), :], 0.0)

    @pl.when(is_first)
    def _wait_w2():
        pltpu.make_async_copy(w2_hbm.at[e], w2_scr, w2_sem).wait()

    y = jnp.dot(h, w2_scr[...], preferred_element_type=jnp.float32)
    y = y + b2_ref[pl.ds(e, 1), :]
    rows = t * TILE + lax.broadcasted_iota(jnp.int32, (TILE, 1), 0)
    mask = (rows >= off_ref[e]) & (rows < off_ref[e + 1])
    out_ref[...] = jnp.where(mask, y, out_ref[...])

    @pl.when(next_e != e)
    def _prefetch_w2():
        pltpu.make_async_copy(w2_hbm.at[next_e], w2_scr, w2_sem).start()


def _grouped_mlp(x_sorted, W1, b1, W2, b2, offsets, tile_ids, group_ids,
                 parity, nxt_e, has_nxt):
    grid_spec = pltpu.PrefetchScalarGridSpec(
        num_scalar_prefetch=6,
        grid=(NUM_ITEMS,),
        in_specs=[
            pl.BlockSpec((TILE, D_MODEL),
                         lambda i, ti, gi, off, par, nx, hn: (ti[i], 0)),
            pl.BlockSpec(memory_space=pltpu.MemorySpace.HBM),
            pl.BlockSpec((N_TASKS, D_FF),
                         lambda i, ti, gi, off, par, nx, hn: (0, 0)),
            pl.BlockSpec(memory_space=pltpu.MemorySpace.HBM),
            pl.BlockSpec((N_TASKS, D_MODEL),
                         lambda i, ti, gi, off, par, nx, hn: (0, 0)),
        ],
        out_specs=pl.BlockSpec((TILE, D_MODEL),
                               lambda i, ti, gi, off, par, nx, hn: (ti[i], 0)),
        scratch_shapes=[
            pltpu.VMEM((2, D_MODEL, D_FF), jnp.float32),
            pltpu.VMEM((D_FF, D_MODEL), jnp.float32),
            pltpu.SemaphoreType.DMA,
            pltpu.SemaphoreType.DMA,
        ],
    )
    return pl.pallas_call(
        _mlp_item,
        grid_spec=grid_spec,
        out_shape=jax.ShapeDtypeStruct((N_TOKENS, D_MODEL), jnp.float32),
        compiler_params=pltpu.CompilerParams(
            dimension_semantics=("arbitrary",),
            vmem_limit_bytes=63 * 1024 * 1024),
    )(tile_ids, group_ids, offsets, parity, nxt_e, has_nxt,
      x_sorted, W1, b1, W2, b2)


_NC = 2
_NS = 16
_NW = _NC * _NS
_BPW = N_TOKENS // _NW
_CH = 16
_NCH = _BPW // _CH


def _sc_gather_rows(table, idx):
    mesh = plsc.VectorSubcoreMesh(core_axis_name="c", subcore_axis_name="s")

    @functools.partial(
        pl.kernel, mesh=mesh,
        out_type=jax.ShapeDtypeStruct((N_TOKENS, D_MODEL), jnp.float32),
        scratch_types=[
            pltpu.VMEM((_BPW,), jnp.int32),
            pltpu.VMEM((2, _CH, D_MODEL), jnp.float32),
            pltpu.SemaphoreType.DMA,
            pltpu.SemaphoreType.DMA,
            pltpu.SemaphoreType.DMA,
            pltpu.SemaphoreType.DMA,
        ],
    )
    def gather_kernel(table_hbm, idx_hbm, out_hbm, idx_v, bufs, g0, g1, w0, w1):
        wid = lax.axis_index("s") * _NC + lax.axis_index("c")
        base = wid * _BPW
        pltpu.sync_copy(idx_hbm.at[pl.ds(base, _BPW)], idx_v)
        gs = (g0, g1)
        ws = (w0, w1)

        def g_copy(c):
            return pltpu.make_async_copy(
                table_hbm.at[idx_v.at[pl.ds(c * _CH, _CH)]],
                bufs.at[c % 2], gs[c % 2])

        def w_copy(c):
            return pltpu.make_async_copy(
                bufs.at[c % 2], out_hbm.at[pl.ds(base + c * _CH, _CH)],
                ws[c % 2])

        g_copy(0).start()
        for c in range(_NCH):
            if c + 1 < _NCH:
                if c >= 1:
                    w_copy(c - 1).wait()
                g_copy(c + 1).start()
            g_copy(c).wait()
            w_copy(c).start()
        w_copy(_NCH - 2).wait()
        w_copy(_NCH - 1).wait()

    return gather_kernel(table, idx)


def kernel(h_root, tasks, W1, b1, W2, b2):
    (sort_idx, inv_idx, offsets, tile_ids, group_ids,
     parity, nxt_e, has_nxt) = _routing(tasks)
    x_sorted = _sc_gather_rows(h_root, sort_idx)
    y_sorted = _grouped_mlp(x_sorted, W1, b1, W2, b2,
                            offsets, tile_ids, group_ids,
                            parity, nxt_e, has_nxt)
    return _sc_gather_rows(y_sorted, inv_idx)
